# baseline (speedup 1.0000x reference)
import jax
import jax.numpy as jnp
from jax.experimental import pallas as pl
from jax.experimental.pallas import tpu as pltpu

_N, _D, _H, _Z, _C = 4096, 512, 256, 128, 10
_G = 64
_RB = 512
_NB = _N // _RB
_F32 = jnp.float32


def _encode_body(x0_ref, x1_ref, eps0_ref, eps1_ref,
                 We0a_ref, be0a_ref, We0b_ref, be0b_ref,
                 Wmu0_ref, bmu0_ref, Wlv0_ref, blv0_ref,
                 We1a_ref, be1a_ref, We1b_ref, be1b_ref,
                 Wmu1_ref, bmu1_ref, Wlv1_ref, blv1_ref,
                 Wg1_ref,
                 mu0_ref, lv0_ref, mu1_ref, lv1_ref, wm_ref, g1_ref):
    def enc(x, Wa, ba, Wb, bb, Wmu, bmu, Wlv, blv):
        h = jnp.maximum(jnp.dot(x, Wa, preferred_element_type=_F32) + ba, 0.0)
        h = jnp.maximum(jnp.dot(h, Wb, preferred_element_type=_F32) + bb, 0.0)
        mu = jnp.dot(h, Wmu, preferred_element_type=_F32) + bmu
        lv = jnp.dot(h, Wlv, preferred_element_type=_F32) + blv
        return mu, lv

    mu0, lv0 = enc(x0_ref[...], We0a_ref[...], be0a_ref[...], We0b_ref[...],
                   be0b_ref[...], Wmu0_ref[...], bmu0_ref[...], Wlv0_ref[...],
                   blv0_ref[...])
    mu1, lv1 = enc(x1_ref[...], We1a_ref[...], be1a_ref[...], We1b_ref[...],
                   be1b_ref[...], Wmu1_ref[...], bmu1_ref[...], Wlv1_ref[...],
                   blv1_ref[...])
    z0 = mu0 + eps0_ref[...] * jnp.exp(0.5 * lv0)
    z1 = mu1 + eps1_ref[...] * jnp.exp(0.5 * lv1)
    joint = 0.5 * (z0 + z1)
    mu0_ref[...] = mu0
    lv0_ref[...] = lv0
    mu1_ref[...] = mu1
    lv1_ref[...] = lv1
    wm_ref[...] = jnp.concatenate([z0, z1], axis=1) * _F32(0.5 ** 0.5)
    g1_ref[...] = jnp.dot(joint, Wg1_ref[...], preferred_element_type=_F32)


def _adj_body(t_ref, theta_ref, wmi_ref, wmj_ref, a_ref, rs_ref):
    i = pl.program_id(0)
    j = pl.program_id(1)
    pd = jax.lax.dot_general(wmi_ref[...], wmj_ref[...],
                             (((1,), (1,)), ((), ())),
                             preferred_element_type=_F32)
    t = t_ref[0, 0]
    th = theta_ref[0, 0]
    a = 1.0 / (1.0 + jnp.exp(-t * (pd + th)))
    row_ids = jax.lax.broadcasted_iota(jnp.int32, (_RB, _RB), 0)
    col_ids = jax.lax.broadcasted_iota(jnp.int32, (_RB, _RB), 1)
    a = a + jnp.where((i == j) & (row_ids == col_ids), _F32(1.0), _F32(0.0))
    a_ref[...] = a.astype(jnp.bfloat16)
    rsum = jnp.sum(a, axis=1, keepdims=True)

    @pl.when(j == 0)
    def _():
        rs_ref[...] = rsum

    @pl.when(j != 0)
    def _():
        rs_ref[...] += rsum


def _mm1_body(a_ref, g1j_ref, rsj_ref, rsi_ref, bg1_ref, wg2_ref,
              z2_ref, acc_ref):
    j = pl.program_id(1)
    xs = g1j_ref[...] * jax.lax.rsqrt(rsj_ref[...])
    contrib = jnp.dot(a_ref[...].astype(_F32), xs,
                      preferred_element_type=_F32)

    @pl.when(j == 0)
    def _():
        acc_ref[...] = contrib

    @pl.when(j != 0)
    def _():
        acc_ref[...] += contrib

    @pl.when(j == _NB - 1)
    def _():
        dinv_i = jax.lax.rsqrt(rsi_ref[...])
        h1 = jnp.maximum(dinv_i * acc_ref[...] + bg1_ref[...], 0.0)
        z2_ref[...] = dinv_i * jnp.dot(h1, wg2_ref[...],
                                       preferred_element_type=_F32)


def _mm2_body(a_ref, z2j_ref, rsi_ref, bg2_ref, pred_ref, acc_ref):
    j = pl.program_id(1)
    contrib = jnp.dot(a_ref[...].astype(_F32), z2j_ref[...],
                      preferred_element_type=_F32)

    @pl.when(j == 0)
    def _():
        acc_ref[...] = contrib

    @pl.when(j != 0)
    def _():
        acc_ref[...] += contrib

    @pl.when(j == _NB - 1)
    def _():
        out = jax.lax.rsqrt(rsi_ref[...]) * acc_ref[...] + bg2_ref[...]
        m = jnp.max(out, axis=-1, keepdims=True)
        lse = jnp.log(jnp.sum(jnp.exp(out - m), axis=-1, keepdims=True)) + m
        pred_ref[...] = out - lse


def kernel(x0, x1, We0a, be0a, We0b, be0b, Wmu0, bmu0, Wlv0, blv0,
           We1a, be1a, We1b, be1b, Wmu1, bmu1, Wlv1, blv1,
           Wg1, bg1, Wg2, bg2, t, theta, eps0, eps1):
    r1 = lambda b: b.reshape(1, -1)
    row = lambda w: pl.BlockSpec((_RB, w), lambda i: (i, 0))
    full = lambda a, b: pl.BlockSpec((a, b), lambda i: (0, 0))

    mu0, lv0, mu1, lv1, wm, g1 = pl.pallas_call(
        _encode_body,
        grid=(_NB,),
        in_specs=[row(_D), row(_D), row(_Z), row(_Z),
                  full(_D, _H), full(1, _H), full(_H, _H), full(1, _H),
                  full(_H, _Z), full(1, _Z), full(_H, _Z), full(1, _Z),
                  full(_D, _H), full(1, _H), full(_H, _H), full(1, _H),
                  full(_H, _Z), full(1, _Z), full(_H, _Z), full(1, _Z),
                  full(_Z, _G)],
        out_specs=[row(_Z), row(_Z), row(_Z), row(_Z), row(2 * _Z), row(_G)],
        out_shape=[jax.ShapeDtypeStruct((_N, _Z), _F32)] * 4 +
                  [jax.ShapeDtypeStruct((_N, 2 * _Z), _F32),
                   jax.ShapeDtypeStruct((_N, _G), _F32)],
        compiler_params=pltpu.CompilerParams(
            dimension_semantics=("parallel",)),
    )(x0, x1, eps0, eps1,
      We0a, r1(be0a), We0b, r1(be0b), Wmu0, r1(bmu0), Wlv0, r1(blv0),
      We1a, r1(be1a), We1b, r1(be1b), Wmu1, r1(bmu1), Wlv1, r1(blv1),
      Wg1)

    rowi = lambda w: pl.BlockSpec((_RB, w), lambda i, j: (i, 0))
    rowj = lambda w: pl.BlockSpec((_RB, w), lambda i, j: (j, 0))
    full2 = lambda a, b: pl.BlockSpec((a, b), lambda i, j: (0, 0))
    ablk = pl.BlockSpec((_RB, _RB), lambda i, j: (i, j))

    aprime, rowsum = pl.pallas_call(
        _adj_body,
        grid=(_NB, _NB),
        in_specs=[full2(1, 1), full2(1, 1), rowi(2 * _Z), rowj(2 * _Z)],
        out_specs=[ablk, rowi(1)],
        out_shape=[jax.ShapeDtypeStruct((_N, _N), jnp.bfloat16),
                   jax.ShapeDtypeStruct((_N, 1), _F32)],
        compiler_params=pltpu.CompilerParams(
            dimension_semantics=("parallel", "arbitrary")),
    )(t.reshape(1, 1), theta.reshape(1, 1), wm, wm)

    z2 = pl.pallas_call(
        _mm1_body,
        grid=(_NB, _NB),
        in_specs=[ablk, rowj(_G), rowj(1), rowi(1),
                  full2(1, _G), full2(_G, _C)],
        out_specs=rowi(_C),
        out_shape=jax.ShapeDtypeStruct((_N, _C), _F32),
        scratch_shapes=[pltpu.VMEM((_RB, _G), _F32)],
        compiler_params=pltpu.CompilerParams(
            dimension_semantics=("parallel", "arbitrary")),
    )(aprime, g1, rowsum, rowsum, r1(bg1), Wg2)

    pred = pl.pallas_call(
        _mm2_body,
        grid=(_NB, _NB),
        in_specs=[ablk, rowj(_C), rowi(1), full2(1, _C)],
        out_specs=rowi(_C),
        out_shape=jax.ShapeDtypeStruct((_N, _C), _F32),
        scratch_shapes=[pltpu.VMEM((_RB, _C), _F32)],
        compiler_params=pltpu.CompilerParams(
            dimension_semantics=("parallel", "arbitrary")),
    )(aprime, z2, rowsum, r1(bg2))

    return pred, mu0, mu1, lv0, lv1


# speedup vs baseline: 1.0428x; 1.0428x over previous
import jax
import jax.numpy as jnp
from jax.experimental import pallas as pl
from jax.experimental.pallas import tpu as pltpu

_N, _D, _H, _Z, _C = 4096, 512, 256, 128, 10
_G = 64
_RB = 512
_NB = _N // _RB
_F32 = jnp.float32


def _encode_body(x0_ref, x1_ref, eps0_ref, eps1_ref,
                 We0a_ref, be0a_ref, We0b_ref, be0b_ref,
                 Wmu0_ref, bmu0_ref, Wlv0_ref, blv0_ref,
                 We1a_ref, be1a_ref, We1b_ref, be1b_ref,
                 Wmu1_ref, bmu1_ref, Wlv1_ref, blv1_ref,
                 Wg1_ref,
                 mu0_ref, lv0_ref, mu1_ref, lv1_ref, wm_ref, g1_ref):
    def enc(x, Wa, ba, Wb, bb, Wmu, bmu, Wlv, blv):
        h = jnp.maximum(jnp.dot(x, Wa, preferred_element_type=_F32) + ba, 0.0)
        h = jnp.maximum(jnp.dot(h, Wb, preferred_element_type=_F32) + bb, 0.0)
        mu = jnp.dot(h, Wmu, preferred_element_type=_F32) + bmu
        lv = jnp.dot(h, Wlv, preferred_element_type=_F32) + blv
        return mu, lv

    mu0, lv0 = enc(x0_ref[...], We0a_ref[...], be0a_ref[...], We0b_ref[...],
                   be0b_ref[...], Wmu0_ref[...], bmu0_ref[...], Wlv0_ref[...],
                   blv0_ref[...])
    mu1, lv1 = enc(x1_ref[...], We1a_ref[...], be1a_ref[...], We1b_ref[...],
                   be1b_ref[...], Wmu1_ref[...], bmu1_ref[...], Wlv1_ref[...],
                   blv1_ref[...])
    z0 = mu0 + eps0_ref[...] * jnp.exp(0.5 * lv0)
    z1 = mu1 + eps1_ref[...] * jnp.exp(0.5 * lv1)
    joint = 0.5 * (z0 + z1)
    mu0_ref[...] = mu0
    lv0_ref[...] = lv0
    mu1_ref[...] = mu1
    lv1_ref[...] = lv1
    wm = jnp.concatenate([z0, z1], axis=1) * _F32(0.5 ** 0.5)
    wm_ref[...] = wm.astype(jnp.bfloat16)
    g1_ref[...] = jnp.dot(joint, Wg1_ref[...], preferred_element_type=_F32)


def _adj_body(t_ref, theta_ref, wmi_ref, wmj_ref, a_ref, rs_ref):
    i = pl.program_id(0)
    j = pl.program_id(1)
    pd = jax.lax.dot_general(wmi_ref[...], wmj_ref[...],
                             (((1,), (1,)), ((), ())),
                             preferred_element_type=_F32)
    t = t_ref[0, 0]
    th = theta_ref[0, 0]
    a = 1.0 / (1.0 + jnp.exp(-t * (pd + th)))
    row_ids = jax.lax.broadcasted_iota(jnp.int32, (_RB, _RB), 0)
    col_ids = jax.lax.broadcasted_iota(jnp.int32, (_RB, _RB), 1)
    a = a + jnp.where((i == j) & (row_ids == col_ids), _F32(1.0), _F32(0.0))
    a_ref[...] = a.astype(jnp.bfloat16)
    rsum = jnp.sum(a, axis=1, keepdims=True)

    @pl.when(j == 0)
    def _():
        rs_ref[...] = rsum

    @pl.when(j != 0)
    def _():
        rs_ref[...] += rsum


def _mm1_body(a_ref, g1j_ref, rsj_ref, rsi_ref, bg1_ref, wg2_ref,
              z2_ref, acc_ref):
    j = pl.program_id(1)
    xs = (g1j_ref[...] * jax.lax.rsqrt(rsj_ref[...])).astype(jnp.bfloat16)
    contrib = jnp.dot(a_ref[...], xs, preferred_element_type=_F32)

    @pl.when(j == 0)
    def _():
        acc_ref[...] = contrib

    @pl.when(j != 0)
    def _():
        acc_ref[...] += contrib

    @pl.when(j == _NB - 1)
    def _():
        dinv_i = jax.lax.rsqrt(rsi_ref[...])
        h1 = jnp.maximum(dinv_i * acc_ref[...] + bg1_ref[...], 0.0)
        z2 = dinv_i * jnp.dot(h1, wg2_ref[...], preferred_element_type=_F32)
        z2_ref[...] = z2.astype(jnp.bfloat16)


def _mm2_body(a_ref, z2j_ref, rsi_ref, bg2_ref, pred_ref, acc_ref):
    j = pl.program_id(1)
    contrib = jnp.dot(a_ref[...], z2j_ref[...], preferred_element_type=_F32)

    @pl.when(j == 0)
    def _():
        acc_ref[...] = contrib

    @pl.when(j != 0)
    def _():
        acc_ref[...] += contrib

    @pl.when(j == _NB - 1)
    def _():
        out = jax.lax.rsqrt(rsi_ref[...]) * acc_ref[...] + bg2_ref[...]
        m = jnp.max(out, axis=-1, keepdims=True)
        lse = jnp.log(jnp.sum(jnp.exp(out - m), axis=-1, keepdims=True)) + m
        pred_ref[...] = out - lse


def kernel(x0, x1, We0a, be0a, We0b, be0b, Wmu0, bmu0, Wlv0, blv0,
           We1a, be1a, We1b, be1b, Wmu1, bmu1, Wlv1, blv1,
           Wg1, bg1, Wg2, bg2, t, theta, eps0, eps1):
    r1 = lambda b: b.reshape(1, -1)
    row = lambda w: pl.BlockSpec((_RB, w), lambda i: (i, 0))
    full = lambda a, b: pl.BlockSpec((a, b), lambda i: (0, 0))

    mu0, lv0, mu1, lv1, wm, g1 = pl.pallas_call(
        _encode_body,
        grid=(_NB,),
        in_specs=[row(_D), row(_D), row(_Z), row(_Z),
                  full(_D, _H), full(1, _H), full(_H, _H), full(1, _H),
                  full(_H, _Z), full(1, _Z), full(_H, _Z), full(1, _Z),
                  full(_D, _H), full(1, _H), full(_H, _H), full(1, _H),
                  full(_H, _Z), full(1, _Z), full(_H, _Z), full(1, _Z),
                  full(_Z, _G)],
        out_specs=[row(_Z), row(_Z), row(_Z), row(_Z), row(2 * _Z), row(_G)],
        out_shape=[jax.ShapeDtypeStruct((_N, _Z), _F32)] * 4 +
                  [jax.ShapeDtypeStruct((_N, 2 * _Z), jnp.bfloat16),
                   jax.ShapeDtypeStruct((_N, _G), _F32)],
        compiler_params=pltpu.CompilerParams(
            dimension_semantics=("parallel",)),
    )(x0, x1, eps0, eps1,
      We0a, r1(be0a), We0b, r1(be0b), Wmu0, r1(bmu0), Wlv0, r1(blv0),
      We1a, r1(be1a), We1b, r1(be1b), Wmu1, r1(bmu1), Wlv1, r1(blv1),
      Wg1)

    rowi = lambda w: pl.BlockSpec((_RB, w), lambda i, j: (i, 0))
    rowj = lambda w: pl.BlockSpec((_RB, w), lambda i, j: (j, 0))
    full2 = lambda a, b: pl.BlockSpec((a, b), lambda i, j: (0, 0))
    ablk = pl.BlockSpec((_RB, _RB), lambda i, j: (i, j))

    aprime, rowsum = pl.pallas_call(
        _adj_body,
        grid=(_NB, _NB),
        in_specs=[full2(1, 1), full2(1, 1), rowi(2 * _Z), rowj(2 * _Z)],
        out_specs=[ablk, rowi(1)],
        out_shape=[jax.ShapeDtypeStruct((_N, _N), jnp.bfloat16),
                   jax.ShapeDtypeStruct((_N, 1), _F32)],
        compiler_params=pltpu.CompilerParams(
            dimension_semantics=("parallel", "arbitrary")),
    )(t.reshape(1, 1), theta.reshape(1, 1), wm, wm)

    z2 = pl.pallas_call(
        _mm1_body,
        grid=(_NB, _NB),
        in_specs=[ablk, rowj(_G), rowj(1), rowi(1),
                  full2(1, _G), full2(_G, _C)],
        out_specs=rowi(_C),
        out_shape=jax.ShapeDtypeStruct((_N, _C), jnp.bfloat16),
        scratch_shapes=[pltpu.VMEM((_RB, _G), _F32)],
        compiler_params=pltpu.CompilerParams(
            dimension_semantics=("parallel", "arbitrary")),
    )(aprime, g1, rowsum, rowsum, r1(bg1), Wg2)

    pred = pl.pallas_call(
        _mm2_body,
        grid=(_NB, _NB),
        in_specs=[ablk, rowj(_C), rowi(1), full2(1, _C)],
        out_specs=rowi(_C),
        out_shape=jax.ShapeDtypeStruct((_N, _C), _F32),
        scratch_shapes=[pltpu.VMEM((_RB, _C), _F32)],
        compiler_params=pltpu.CompilerParams(
            dimension_semantics=("parallel", "arbitrary")),
    )(aprime, z2, rowsum, r1(bg2))

    return pred, mu0, mu1, lv0, lv1


# speedup vs baseline: 1.3797x; 1.3231x over previous
import jax
import jax.numpy as jnp
from jax.experimental import pallas as pl
from jax.experimental.pallas import tpu as pltpu

_N, _D, _H, _Z, _C = 4096, 512, 256, 128, 10
_G = 64
_RB = 512
_NB = _N // _RB
_CB = 1024
_NC = _N // _CB
_F32 = jnp.float32
_BF16 = jnp.bfloat16


def _encode_body(x0_ref, x1_ref, eps0_ref, eps1_ref,
                 We0a_ref, be0a_ref, We0b_ref, be0b_ref,
                 Wmu0_ref, bmu0_ref, Wlv0_ref, blv0_ref,
                 We1a_ref, be1a_ref, We1b_ref, be1b_ref,
                 Wmu1_ref, bmu1_ref, Wlv1_ref, blv1_ref,
                 Wg1_ref,
                 mu0_ref, lv0_ref, mu1_ref, lv1_ref, wm_ref, g1_ref):
    def enc(x, Wa, ba, Wb, bb, Wmu, bmu, Wlv, blv):
        h = jnp.maximum(jnp.dot(x, Wa, preferred_element_type=_F32) + ba, 0.0)
        h = jnp.maximum(jnp.dot(h, Wb, preferred_element_type=_F32) + bb, 0.0)
        mu = jnp.dot(h, Wmu, preferred_element_type=_F32) + bmu
        lv = jnp.dot(h, Wlv, preferred_element_type=_F32) + blv
        return mu, lv

    mu0, lv0 = enc(x0_ref[...], We0a_ref[...], be0a_ref[...], We0b_ref[...],
                   be0b_ref[...], Wmu0_ref[...], bmu0_ref[...], Wlv0_ref[...],
                   blv0_ref[...])
    mu1, lv1 = enc(x1_ref[...], We1a_ref[...], be1a_ref[...], We1b_ref[...],
                   be1b_ref[...], Wmu1_ref[...], bmu1_ref[...], Wlv1_ref[...],
                   blv1_ref[...])
    z0 = mu0 + eps0_ref[...] * jnp.exp(0.5 * lv0)
    z1 = mu1 + eps1_ref[...] * jnp.exp(0.5 * lv1)
    joint = 0.5 * (z0 + z1)
    mu0_ref[...] = mu0
    lv0_ref[...] = lv0
    mu1_ref[...] = mu1
    lv1_ref[...] = lv1
    wm = jnp.concatenate([z0, z1], axis=1) * _F32(0.5 ** 0.5)
    wm_ref[...] = wm.astype(_BF16)
    g1_ref[...] = jnp.dot(joint, Wg1_ref[...], preferred_element_type=_F32)


def _graph_body(t_ref, theta_ref, wm_ref, g1_ref, bg1_ref, wg2_ref, bg2_ref,
                pred_ref, rs_ref, z2_ref):
    p = pl.program_id(0)
    i = pl.program_id(1)
    t = t_ref[0, 0]
    th = theta_ref[0, 0]
    wmi = wm_ref[pl.ds(i * _RB, _RB), :]

    def a_block(jc):
        wmj = wm_ref[pl.ds(jc * _CB, _CB), :]
        pd = jax.lax.dot_general(wmi, wmj, (((1,), (1,)), ((), ())),
                                 preferred_element_type=_F32)
        a = 1.0 / (1.0 + jnp.exp(-t * (pd + th)))
        rows = jax.lax.broadcasted_iota(jnp.int32, (_RB, _CB), 0) + i * _RB
        cols = jax.lax.broadcasted_iota(jnp.int32, (_RB, _CB), 1) + jc * _CB
        return a + jnp.where(rows == cols, _F32(1.0), _F32(0.0))

    @pl.when(p == 0)
    def _():
        def body(jc, acc):
            return acc + jnp.sum(a_block(jc), axis=1, keepdims=True)
        rs = jax.lax.fori_loop(0, _NC, body, jnp.zeros((_RB, 1), _F32))
        rs_ref[pl.ds(i * _RB, _RB), :] = rs

    @pl.when(p == 1)
    def _():
        def body(jc, acc):
            a = a_block(jc).astype(_BF16)
            g1j = g1_ref[pl.ds(jc * _CB, _CB), :]
            rsj = rs_ref[pl.ds(jc * _CB, _CB), :]
            xs = (g1j * jax.lax.rsqrt(rsj)).astype(_BF16)
            return acc + jnp.dot(a, xs, preferred_element_type=_F32)
        acc = jax.lax.fori_loop(0, _NC, body, jnp.zeros((_RB, _G), _F32))
        dinv = jax.lax.rsqrt(rs_ref[pl.ds(i * _RB, _RB), :])
        h1 = jnp.maximum(dinv * acc + bg1_ref[...], 0.0)
        z2_ref[pl.ds(i * _RB, _RB), :] = dinv * jnp.dot(
            h1, wg2_ref[...], preferred_element_type=_F32)

    @pl.when(p == 2)
    def _():
        def body(jc, acc):
            a = a_block(jc).astype(_BF16)
            zj = z2_ref[pl.ds(jc * _CB, _CB), :].astype(_BF16)
            return acc + jnp.dot(a, zj, preferred_element_type=_F32)
        acc = jax.lax.fori_loop(0, _NC, body, jnp.zeros((_RB, _C), _F32))
        out = jax.lax.rsqrt(rs_ref[pl.ds(i * _RB, _RB), :]) * acc + bg2_ref[...]
        m = jnp.max(out, axis=-1, keepdims=True)
        lse = jnp.log(jnp.sum(jnp.exp(out - m), axis=-1, keepdims=True)) + m
        pred_ref[pl.ds(i * _RB, _RB), :] = out - lse


def kernel(x0, x1, We0a, be0a, We0b, be0b, Wmu0, bmu0, Wlv0, blv0,
           We1a, be1a, We1b, be1b, Wmu1, bmu1, Wlv1, blv1,
           Wg1, bg1, Wg2, bg2, t, theta, eps0, eps1):
    r1 = lambda b: b.reshape(1, -1)
    row = lambda w: pl.BlockSpec((_RB, w), lambda i: (i, 0))
    full = lambda a, b: pl.BlockSpec((a, b), lambda i: (0, 0))

    mu0, lv0, mu1, lv1, wm, g1 = pl.pallas_call(
        _encode_body,
        grid=(_NB,),
        in_specs=[row(_D), row(_D), row(_Z), row(_Z),
                  full(_D, _H), full(1, _H), full(_H, _H), full(1, _H),
                  full(_H, _Z), full(1, _Z), full(_H, _Z), full(1, _Z),
                  full(_D, _H), full(1, _H), full(_H, _H), full(1, _H),
                  full(_H, _Z), full(1, _Z), full(_H, _Z), full(1, _Z),
                  full(_Z, _G)],
        out_specs=[row(_Z), row(_Z), row(_Z), row(_Z), row(2 * _Z), row(_G)],
        out_shape=[jax.ShapeDtypeStruct((_N, _Z), _F32)] * 4 +
                  [jax.ShapeDtypeStruct((_N, 2 * _Z), _BF16),
                   jax.ShapeDtypeStruct((_N, _G), _F32)],
        compiler_params=pltpu.CompilerParams(
            dimension_semantics=("parallel",)),
    )(x0, x1, eps0, eps1,
      We0a, r1(be0a), We0b, r1(be0b), Wmu0, r1(bmu0), Wlv0, r1(blv0),
      We1a, r1(be1a), We1b, r1(be1b), Wmu1, r1(bmu1), Wlv1, r1(blv1),
      Wg1)

    full2 = lambda a, b: pl.BlockSpec((a, b), lambda p, i: (0, 0))

    pred = pl.pallas_call(
        _graph_body,
        grid=(3, _NB),
        in_specs=[full2(1, 1), full2(1, 1), full2(_N, 2 * _Z), full2(_N, _G),
                  full2(1, _G), full2(_G, _C), full2(1, _C)],
        out_specs=pl.BlockSpec((_N, _C), lambda p, i: (0, 0)),
        out_shape=jax.ShapeDtypeStruct((_N, _C), _F32),
        scratch_shapes=[pltpu.VMEM((_N, 1), _F32),
                        pltpu.VMEM((_N, _C), _F32)],
        compiler_params=pltpu.CompilerParams(
            dimension_semantics=("arbitrary", "arbitrary")),
    )(t.reshape(1, 1), theta.reshape(1, 1), wm, g1, r1(bg1), Wg2, r1(bg2))

    return pred, mu0, mu1, lv0, lv1


# speedup vs baseline: 2.6495x; 1.9203x over previous
import jax
import jax.numpy as jnp
from jax.experimental import pallas as pl
from jax.experimental.pallas import tpu as pltpu

_N, _D, _H, _Z, _C = 4096, 512, 256, 128, 10
_G = 64
_RB = 512
_NB = _N // _RB
_CB = 1024
_NC = _N // _CB
_F32 = jnp.float32
_BF16 = jnp.bfloat16


def _encode_body(x0_ref, x1_ref, eps0_ref, eps1_ref,
                 We0a_ref, be0a_ref, We0b_ref, be0b_ref,
                 Wmu0_ref, bmu0_ref, Wlv0_ref, blv0_ref,
                 We1a_ref, be1a_ref, We1b_ref, be1b_ref,
                 Wmu1_ref, bmu1_ref, Wlv1_ref, blv1_ref,
                 Wg1_ref,
                 mu0_ref, lv0_ref, mu1_ref, lv1_ref, wm_ref, g1_ref):
    def enc(x, Wa, ba, Wb, bb, Wmu, bmu, Wlv, blv):
        h = jnp.maximum(jnp.dot(x, Wa, preferred_element_type=_F32) + ba, 0.0)
        h = jnp.maximum(jnp.dot(h, Wb, preferred_element_type=_F32) + bb, 0.0)
        mu = jnp.dot(h, Wmu, preferred_element_type=_F32) + bmu
        lv = jnp.dot(h, Wlv, preferred_element_type=_F32) + blv
        return mu, lv

    mu0, lv0 = enc(x0_ref[...], We0a_ref[...], be0a_ref[...], We0b_ref[...],
                   be0b_ref[...], Wmu0_ref[...], bmu0_ref[...], Wlv0_ref[...],
                   blv0_ref[...])
    mu1, lv1 = enc(x1_ref[...], We1a_ref[...], be1a_ref[...], We1b_ref[...],
                   be1b_ref[...], Wmu1_ref[...], bmu1_ref[...], Wlv1_ref[...],
                   blv1_ref[...])
    z0 = mu0 + eps0_ref[...] * jnp.exp(0.5 * lv0)
    z1 = mu1 + eps1_ref[...] * jnp.exp(0.5 * lv1)
    joint = 0.5 * (z0 + z1)
    mu0_ref[...] = mu0
    lv0_ref[...] = lv0
    mu1_ref[...] = mu1
    lv1_ref[...] = lv1
    wm = jnp.concatenate([z0, z1], axis=1) * _F32(0.5 ** 0.5)
    wm_ref[...] = wm.astype(_BF16)
    g1_ref[...] = jnp.dot(joint, Wg1_ref[...], preferred_element_type=_F32)


def _graph_body(t_ref, theta_ref, wm_ref, g1_ref, bg1_ref, wg2_ref, bg2_ref,
                pred_ref, rs_ref, z2_ref, a_ref):
    p = pl.program_id(0)
    i = pl.program_id(1)
    t = t_ref[0, 0]
    th = theta_ref[0, 0]
    ri = pl.ds(i * _RB, _RB)

    @pl.when(p == 0)
    def _():
        wmi = wm_ref[ri, :]

        def body(jc, acc):
            cj = pl.ds(jc * _CB, _CB)
            wmj = wm_ref[cj, :]
            pd = jax.lax.dot_general(wmi, wmj, (((1,), (1,)), ((), ())),
                                     preferred_element_type=_F32)
            a = 0.5 * jnp.tanh((0.5 * t) * (pd + th)) + 0.5
            a_ref[ri, cj] = a.astype(_BF16)
            return acc + jnp.sum(a, axis=1, keepdims=True)

        rs = jax.lax.fori_loop(0, _NC, body, jnp.full((_RB, 1), 1.0, _F32))
        rs_ref[ri, :] = rs

    @pl.when(p == 1)
    def _():
        xs32 = g1_ref[...] * jax.lax.rsqrt(rs_ref[...])
        acc = jnp.dot(a_ref[ri, :], xs32.astype(_BF16),
                      preferred_element_type=_F32)
        dinv = jax.lax.rsqrt(rs_ref[ri, :])
        acc = acc + g1_ref[ri, :] * dinv
        h1 = jnp.maximum(dinv * acc + bg1_ref[...], 0.0)
        z2_ref[ri, :] = dinv * jnp.dot(h1, wg2_ref[...],
                                       preferred_element_type=_F32)

    @pl.when(p == 2)
    def _():
        z2 = z2_ref[...].astype(_BF16)
        acc = jnp.dot(a_ref[ri, :], z2, preferred_element_type=_F32)
        acc = acc + z2_ref[ri, :]
        out = jax.lax.rsqrt(rs_ref[ri, :]) * acc + bg2_ref[...]
        m = jnp.max(out, axis=-1, keepdims=True)
        lse = jnp.log(jnp.sum(jnp.exp(out - m), axis=-1, keepdims=True)) + m
        pred_ref[ri, :] = out - lse


def kernel(x0, x1, We0a, be0a, We0b, be0b, Wmu0, bmu0, Wlv0, blv0,
           We1a, be1a, We1b, be1b, Wmu1, bmu1, Wlv1, blv1,
           Wg1, bg1, Wg2, bg2, t, theta, eps0, eps1):
    r1 = lambda b: b.reshape(1, -1)
    row = lambda w: pl.BlockSpec((_RB, w), lambda i: (i, 0))
    full = lambda a, b: pl.BlockSpec((a, b), lambda i: (0, 0))

    mu0, lv0, mu1, lv1, wm, g1 = pl.pallas_call(
        _encode_body,
        grid=(_NB,),
        in_specs=[row(_D), row(_D), row(_Z), row(_Z),
                  full(_D, _H), full(1, _H), full(_H, _H), full(1, _H),
                  full(_H, _Z), full(1, _Z), full(_H, _Z), full(1, _Z),
                  full(_D, _H), full(1, _H), full(_H, _H), full(1, _H),
                  full(_H, _Z), full(1, _Z), full(_H, _Z), full(1, _Z),
                  full(_Z, _G)],
        out_specs=[row(_Z), row(_Z), row(_Z), row(_Z), row(2 * _Z), row(_G)],
        out_shape=[jax.ShapeDtypeStruct((_N, _Z), _F32)] * 4 +
                  [jax.ShapeDtypeStruct((_N, 2 * _Z), _BF16),
                   jax.ShapeDtypeStruct((_N, _G), _F32)],
        compiler_params=pltpu.CompilerParams(
            dimension_semantics=("parallel",)),
    )(x0, x1, eps0, eps1,
      We0a, r1(be0a), We0b, r1(be0b), Wmu0, r1(bmu0), Wlv0, r1(blv0),
      We1a, r1(be1a), We1b, r1(be1b), Wmu1, r1(bmu1), Wlv1, r1(blv1),
      Wg1)

    full2 = lambda a, b: pl.BlockSpec((a, b), lambda p, i: (0, 0))

    pred = pl.pallas_call(
        _graph_body,
        grid=(3, _NB),
        in_specs=[full2(1, 1), full2(1, 1), full2(_N, 2 * _Z), full2(_N, _G),
                  full2(1, _G), full2(_G, _C), full2(1, _C)],
        out_specs=pl.BlockSpec((_N, _C), lambda p, i: (0, 0)),
        out_shape=jax.ShapeDtypeStruct((_N, _C), _F32),
        scratch_shapes=[pltpu.VMEM((_N, 1), _F32),
                        pltpu.VMEM((_N, _C), _F32),
                        pltpu.VMEM((_N, _N), _BF16)],
        compiler_params=pltpu.CompilerParams(
            dimension_semantics=("arbitrary", "arbitrary")),
    )(t.reshape(1, 1), theta.reshape(1, 1), wm, g1, r1(bg1), Wg2, r1(bg2))

    return pred, mu0, mu1, lv0, lv1
